# Initial kernel scaffold; baseline (speedup 1.0000x reference)
import jax
import jax.numpy as jnp
from jax.experimental import pallas as pl


def kernel(input, output, input_mask, output_mask, input_word_len, output_word_len, emb, w_ih_enc, w_hh_enc, b_ih_enc, b_hh_enc, w_ih_dec, w_hh_dec, b_ih_dec, b_hh_dec):
    raise NotImplementedError("write your pallas kernel here")


# speedup vs baseline: 3.7998x; 3.7998x over previous
import jax
import jax.numpy as jnp
from jax.experimental import pallas as pl

B, L, V, E, H = 16384, 20, 512, 64, 64
BLK = 1024
NB = B // BLK


def _gru_gates(gi, gh, h):
    r = jax.nn.sigmoid(gi[:, :H] + gh[:, :H])
    z = jax.nn.sigmoid(gi[:, H:2 * H] + gh[:, H:2 * H])
    n = jnp.tanh(gi[:, 2 * H:] + r * gh[:, 2 * H:])
    return (1.0 - z) * n + z * h


def _body(ids_e, mask_e, ids_d, mask_d, emb, wie, whe, bie, bhe,
          wid, whd, bid, bhd, out):
    iota = jax.lax.broadcasted_iota(jnp.int32, (BLK, V), 1)
    embv = emb[:]

    def step(ids_ref, mask_ref, h, t, wi, wh, bi, bh):
        idc = ids_ref[:, t:t + 1]
        oh = jnp.where(idc == iota, 1.0, 0.0)
        x = jax.lax.dot_general(oh, embv, (((1,), (0,)), ((), ())),
                                preferred_element_type=jnp.float32)
        gi = jax.lax.dot_general(x, wi, (((1,), (1,)), ((), ())),
                                 preferred_element_type=jnp.float32) + bi
        gh = jax.lax.dot_general(h, wh, (((1,), (1,)), ((), ())),
                                 preferred_element_type=jnp.float32) + bh
        h_new = _gru_gates(gi, gh, h)
        m = mask_ref[:, t:t + 1]
        return h_new, m

    wie_, whe_, bie_, bhe_ = wie[:], whe[:], bie[:], bhe[:]
    wid_, whd_, bid_, bhd_ = wid[:], whd[:], bid[:], bhd[:]

    h = jnp.zeros((BLK, H), jnp.float32)
    for t in range(L):
        h_new, m = step(ids_e, mask_e, h, t, wie_, whe_, bie_, bhe_)
        h = m * h_new + (1.0 - m) * h
    for t in range(L):
        h_new, m = step(ids_d, mask_d, h, t, wid_, whd_, bid_, bhd_)
        y = m * h_new
        out[:, t * H:(t + 1) * H] = y
        h = y + (1.0 - m) * h


def kernel(input, output, input_mask, output_mask, input_word_len,
           output_word_len, emb, w_ih_enc, w_hh_enc, b_ih_enc, b_hh_enc,
           w_ih_dec, w_hh_dec, b_ih_dec, b_hh_dec):
    in_len = input_word_len[:, 0]
    perm_in = jnp.argsort(-in_len)
    ids_e = jnp.take(input, perm_in, axis=0).astype(jnp.int32)
    len_e = jnp.take(in_len, perm_in)
    mask_e = (jnp.arange(L)[None, :] < len_e[:, None]).astype(jnp.float32)

    out_len = output_word_len[:, 0]
    perm_out = jnp.argsort(-out_len)
    ids_d = jnp.take(output, perm_out, axis=0).astype(jnp.int32)
    len_d = jnp.take(out_len, perm_out)
    mask_d = (jnp.arange(L)[None, :] < len_d[:, None]).astype(jnp.float32)

    full = lambda shape: pl.BlockSpec(shape, lambda i: (0,) * len(shape))
    blocked = pl.BlockSpec((BLK, L), lambda i: (i, 0))

    out_flat = pl.pallas_call(
        _body,
        grid=(NB,),
        in_specs=[
            blocked, blocked, blocked, blocked,
            full((V, E)),
            full((3 * H, E)), full((3 * H, H)), full((1, 3 * H)), full((1, 3 * H)),
            full((3 * H, E)), full((3 * H, H)), full((1, 3 * H)), full((1, 3 * H)),
        ],
        out_specs=pl.BlockSpec((BLK, L * H), lambda i: (i, 0)),
        out_shape=jax.ShapeDtypeStruct((B, L * H), jnp.float32),
    )(ids_e, mask_e, ids_d, mask_d, emb,
      w_ih_enc, w_hh_enc, b_ih_enc.reshape(1, 3 * H), b_hh_enc.reshape(1, 3 * H),
      w_ih_dec, w_hh_dec, b_ih_dec.reshape(1, 3 * H), b_hh_dec.reshape(1, 3 * H))
    return out_flat.reshape(B, L, H)


# speedup vs baseline: 4.3744x; 1.1512x over previous
import jax
import jax.numpy as jnp
from jax.experimental import pallas as pl
from jax.experimental.pallas import tpu as pltpu

B, L, V, E, H = 16384, 20, 512, 64, 64
BLK = 1024
NB = B // BLK


def _body(ids_e, mask_e, ids_d, mask_d, emb, wie, whe, bie,
          wid, whd, bid, bhe, bhd, out, ge_ref, gd_ref):
    i = pl.program_id(0)

    @pl.when(i == 0)
    def _build_tables():
        embv = emb[:].astype(jnp.bfloat16)
        ge_ref[:] = (jax.lax.dot_general(
            embv, wie[:].astype(jnp.bfloat16), (((1,), (1,)), ((), ())),
            preferred_element_type=jnp.float32) + bie[:]).astype(jnp.bfloat16)
        gd_ref[:] = (jax.lax.dot_general(
            embv, wid[:].astype(jnp.bfloat16), (((1,), (1,)), ((), ())),
            preferred_element_type=jnp.float32) + bid[:]).astype(jnp.bfloat16)

    iota = jax.lax.broadcasted_iota(jnp.int32, (BLK, V), 1)

    def step(ids_ref, mask_ref, g_ref, h, t, wh, bh):
        idc = ids_ref[:, t:t + 1]
        oh = jnp.where(idc == iota, 1.0, 0.0).astype(jnp.bfloat16)
        gi = jax.lax.dot_general(oh, g_ref[:], (((1,), (0,)), ((), ())),
                                 preferred_element_type=jnp.float32)
        gh = jax.lax.dot_general(h.astype(jnp.bfloat16), wh,
                                 (((1,), (1,)), ((), ())),
                                 preferred_element_type=jnp.float32) + bh
        rz = jax.nn.sigmoid(gi[:, :2 * H] + gh[:, :2 * H])
        r = rz[:, :H]
        z = rz[:, H:]
        n = jnp.tanh(gi[:, 2 * H:] + r * gh[:, 2 * H:])
        h_new = (1.0 - z) * n + z * h
        m = mask_ref[:, t:t + 1]
        return h_new, m

    whe_ = whe[:].astype(jnp.bfloat16)
    whd_ = whd[:].astype(jnp.bfloat16)
    bhe_, bhd_ = bhe[:], bhd[:]

    h = jnp.zeros((BLK, H), jnp.float32)
    for t in range(L):
        h_new, m = step(ids_e, mask_e, ge_ref, h, t, whe_, bhe_)
        h = m * h_new + (1.0 - m) * h
    for t in range(L):
        h_new, m = step(ids_d, mask_d, gd_ref, h, t, whd_, bhd_)
        y = m * h_new
        out[:, t * H:(t + 1) * H] = y
        h = y + (1.0 - m) * h


def kernel(input, output, input_mask, output_mask, input_word_len,
           output_word_len, emb, w_ih_enc, w_hh_enc, b_ih_enc, b_hh_enc,
           w_ih_dec, w_hh_dec, b_ih_dec, b_hh_dec):
    in_len = input_word_len[:, 0]
    perm_in = jnp.argsort(-in_len)
    ids_e = jnp.take(input, perm_in, axis=0).astype(jnp.int32)
    len_e = jnp.take(in_len, perm_in)
    mask_e = (jnp.arange(L)[None, :] < len_e[:, None]).astype(jnp.float32)

    out_len = output_word_len[:, 0]
    perm_out = jnp.argsort(-out_len)
    ids_d = jnp.take(output, perm_out, axis=0).astype(jnp.int32)
    len_d = jnp.take(out_len, perm_out)
    mask_d = (jnp.arange(L)[None, :] < len_d[:, None]).astype(jnp.float32)

    full = lambda shape: pl.BlockSpec(shape, lambda i: (0,) * len(shape))
    blocked = pl.BlockSpec((BLK, L), lambda i: (i, 0))

    out_flat = pl.pallas_call(
        _body,
        grid=(NB,),
        in_specs=[
            blocked, blocked, blocked, blocked,
            full((V, E)),
            full((3 * H, E)), full((3 * H, H)), full((1, 3 * H)),
            full((3 * H, E)), full((3 * H, H)), full((1, 3 * H)),
            full((1, 3 * H)), full((1, 3 * H)),
        ],
        out_specs=pl.BlockSpec((BLK, L * H), lambda i: (i, 0)),
        out_shape=jax.ShapeDtypeStruct((B, L * H), jnp.float32),
        scratch_shapes=[
            pltpu.VMEM((V, 3 * H), jnp.bfloat16),
            pltpu.VMEM((V, 3 * H), jnp.bfloat16),
        ],
    )(ids_e, mask_e, ids_d, mask_d, emb,
      w_ih_enc, w_hh_enc, b_ih_enc.reshape(1, 3 * H),
      w_ih_dec, w_hh_dec, b_ih_dec.reshape(1, 3 * H),
      b_hh_enc.reshape(1, 3 * H), b_hh_dec.reshape(1, 3 * H))
    return out_flat.reshape(B, L, H)


# speedup vs baseline: 5.4406x; 1.2437x over previous
import jax
import jax.numpy as jnp
from jax.experimental import pallas as pl
from jax.experimental.pallas import tpu as pltpu

B, L, V, E, H = 16384, 20, 512, 64, 64
BLK = 1024
NB = B // BLK


def _body(lens, ids_e, mask_e, ids_d, mask_d, emb, wie, whe, bie,
          wid, whd, bid, bhe, bhd, out, ge_ref, gd_ref, h_ref):
    i = pl.program_id(0)
    me = lens[i, 0]
    md = lens[i, 1]

    @pl.when(i == 0)
    def _build_tables():
        embv = emb[:].astype(jnp.bfloat16)
        ge_ref[:] = (jax.lax.dot_general(
            embv, wie[:].astype(jnp.bfloat16), (((1,), (1,)), ((), ())),
            preferred_element_type=jnp.float32) + bie[:]).astype(jnp.bfloat16)
        gd_ref[:] = (jax.lax.dot_general(
            embv, wid[:].astype(jnp.bfloat16), (((1,), (1,)), ((), ())),
            preferred_element_type=jnp.float32) + bid[:]).astype(jnp.bfloat16)

    iota = jax.lax.broadcasted_iota(jnp.int32, (BLK, V), 1)

    def step(ids_ref, mask_ref, g_ref, t, wh, bh):
        h = h_ref[:]
        idc = ids_ref[:, t:t + 1]
        oh = jnp.where(idc == iota, 1.0, 0.0).astype(jnp.bfloat16)
        gi = jax.lax.dot_general(oh, g_ref[:], (((1,), (0,)), ((), ())),
                                 preferred_element_type=jnp.float32)
        gh = jax.lax.dot_general(h.astype(jnp.bfloat16), wh,
                                 (((1,), (1,)), ((), ())),
                                 preferred_element_type=jnp.float32) + bh
        rz = jax.nn.sigmoid(gi[:, :2 * H] + gh[:, :2 * H])
        r = rz[:, :H]
        z = rz[:, H:]
        n = jnp.tanh(gi[:, 2 * H:] + r * gh[:, 2 * H:])
        h_new = (1.0 - z) * n + z * h
        m = mask_ref[:, t:t + 1]
        return h, h_new, m

    whe_ = whe[:].astype(jnp.bfloat16)
    whd_ = whd[:].astype(jnp.bfloat16)
    bhe_, bhd_ = bhe[:], bhd[:]

    h_ref[:] = jnp.zeros((BLK, H), jnp.float32)

    def enc_step(t):
        @pl.when(t < me)
        def _():
            h, h_new, m = step(ids_e, mask_e, ge_ref, t, whe_, bhe_)
            h_ref[:] = m * h_new + (1.0 - m) * h

    for t in range(L):
        enc_step(t)

    def dec_step(t):
        @pl.when(t < md)
        def _():
            h, h_new, m = step(ids_d, mask_d, gd_ref, t, whd_, bhd_)
            y = m * h_new
            out[:, t * H:(t + 1) * H] = y
            h_ref[:] = y + (1.0 - m) * h

        @pl.when(t >= md)
        def _():
            out[:, t * H:(t + 1) * H] = jnp.zeros((BLK, H), jnp.float32)

    for t in range(L):
        dec_step(t)


def kernel(input, output, input_mask, output_mask, input_word_len,
           output_word_len, emb, w_ih_enc, w_hh_enc, b_ih_enc, b_hh_enc,
           w_ih_dec, w_hh_dec, b_ih_dec, b_hh_dec):
    in_len = input_word_len[:, 0]
    perm_in = jnp.argsort(-in_len)
    ids_e = jnp.take(input, perm_in, axis=0).astype(jnp.int32)
    len_e = jnp.take(in_len, perm_in)
    mask_e = (jnp.arange(L)[None, :] < len_e[:, None]).astype(jnp.float32)

    out_len = output_word_len[:, 0]
    perm_out = jnp.argsort(-out_len)
    ids_d = jnp.take(output, perm_out, axis=0).astype(jnp.int32)
    len_d = jnp.take(out_len, perm_out)
    mask_d = (jnp.arange(L)[None, :] < len_d[:, None]).astype(jnp.float32)

    maxlens = jnp.stack([len_e[::BLK], len_d[::BLK]], axis=1).astype(jnp.int32)

    full = lambda shape: pl.BlockSpec(shape, lambda i, *_: (0,) * len(shape))
    blocked = pl.BlockSpec((BLK, L), lambda i, *_: (i, 0))

    grid_spec = pltpu.PrefetchScalarGridSpec(
        num_scalar_prefetch=1,
        grid=(NB,),
        in_specs=[
            blocked, blocked, blocked, blocked,
            full((V, E)),
            full((3 * H, E)), full((3 * H, H)), full((1, 3 * H)),
            full((3 * H, E)), full((3 * H, H)), full((1, 3 * H)),
            full((1, 3 * H)), full((1, 3 * H)),
        ],
        out_specs=pl.BlockSpec((BLK, L * H), lambda i, *_: (i, 0)),
        scratch_shapes=[
            pltpu.VMEM((V, 3 * H), jnp.bfloat16),
            pltpu.VMEM((V, 3 * H), jnp.bfloat16),
            pltpu.VMEM((BLK, H), jnp.float32),
        ],
    )

    out_flat = pl.pallas_call(
        _body,
        grid_spec=grid_spec,
        out_shape=jax.ShapeDtypeStruct((B, L * H), jnp.float32),
    )(maxlens, ids_e, mask_e, ids_d, mask_d, emb,
      w_ih_enc, w_hh_enc, b_ih_enc.reshape(1, 3 * H),
      w_ih_dec, w_hh_dec, b_ih_dec.reshape(1, 3 * H),
      b_hh_enc.reshape(1, 3 * H), b_hh_dec.reshape(1, 3 * H))
    return out_flat.reshape(B, L, H)


# speedup vs baseline: 7.9496x; 1.4612x over previous
import jax
import jax.numpy as jnp
from jax.experimental import pallas as pl
from jax.experimental.pallas import tpu as pltpu

B, L, V, E, H = 16384, 20, 512, 64, 64
BLK = 2048
SUB = 1024
NB = B // BLK


def _body(lens, ids_e, len_e, ids_d, len_d, emb, wie, whe, bie,
          wid, whd, bid, bhe, bhd, out, ge_ref, gd_ref, h_ref):
    i = pl.program_id(0)
    me = lens[i, 0]
    md = lens[i, 1]

    @pl.when(i == 0)
    def _build_tables():
        embv = emb[:].astype(jnp.bfloat16)
        ge_ref[:] = (jax.lax.dot_general(
            embv, wie[:].astype(jnp.bfloat16), (((1,), (1,)), ((), ())),
            preferred_element_type=jnp.float32) + bie[:]).astype(jnp.bfloat16)
        gd_ref[:] = (jax.lax.dot_general(
            embv, wid[:].astype(jnp.bfloat16), (((1,), (1,)), ((), ())),
            preferred_element_type=jnp.float32) + bid[:]).astype(jnp.bfloat16)

    iota16 = jax.lax.broadcasted_iota(jnp.int32, (SUB, V), 1).astype(jnp.int16)
    one_bf = jnp.ones((), jnp.bfloat16)
    zero_bf = jnp.zeros((), jnp.bfloat16)

    whe_ = whe[:].astype(jnp.bfloat16)
    whd_ = whd[:].astype(jnp.bfloat16)
    bheb = jnp.broadcast_to(bhe[:], (SUB, 3 * H))
    bhdb = jnp.broadcast_to(bhd[:], (SUB, 3 * H))

    lebA = jnp.broadcast_to(len_e[0:SUB, :], (SUB, H))
    lebB = jnp.broadcast_to(len_e[SUB:BLK, :], (SUB, H))
    ldbA = jnp.broadcast_to(len_d[0:SUB, :], (SUB, H))
    ldbB = jnp.broadcast_to(len_d[SUB:BLK, :], (SUB, H))

    def cell(ids_ref, g_ref, a0, t, wh, bhb, h):
        idc = ids_ref[a0:a0 + SUB, t:t + 1]
        oh = jnp.where(idc == iota16, one_bf, zero_bf)
        gi = jax.lax.dot_general(oh, g_ref[:], (((1,), (0,)), ((), ())),
                                 preferred_element_type=jnp.float32)
        gh = jax.lax.dot_general(h.astype(jnp.bfloat16), wh,
                                 (((1,), (1,)), ((), ())),
                                 preferred_element_type=jnp.float32) + bhb
        rz = jax.nn.sigmoid(gi[:, :2 * H] + gh[:, :2 * H])
        z = rz[:, H:]
        n = jnp.tanh(gi[:, 2 * H:] + rz[:, :H] * gh[:, 2 * H:])
        return (1.0 - z) * n + z * h

    def enc_step(t):
        @pl.when(t < me)
        def _():
            for a0, leb in ((0, lebA), (SUB, lebB)):
                h = h_ref[a0:a0 + SUB, :]
                h_new = cell(ids_e, ge_ref, a0, t, whe_, bheb, h)
                h_ref[a0:a0 + SUB, :] = jnp.where(leb > t, h_new, h)

    def dec_step(t):
        @pl.when(t < md)
        def _():
            for a0, ldb in ((0, ldbA), (SUB, ldbB)):
                h = h_ref[a0:a0 + SUB, :]
                h_new = cell(ids_d, gd_ref, a0, t, whd_, bhdb, h)
                msk = ldb > t
                out[a0:a0 + SUB, t * H:(t + 1) * H] = jnp.where(
                    msk, h_new, 0.0)
                h_ref[a0:a0 + SUB, :] = jnp.where(msk, h_new, h)

        @pl.when(t >= md)
        def _():
            out[:, t * H:(t + 1) * H] = jnp.zeros((BLK, H), jnp.float32)

    h_ref[:] = jnp.zeros((BLK, H), jnp.float32)
    for t in range(L):
        enc_step(t)
    for t in range(L):
        dec_step(t)


def kernel(input, output, input_mask, output_mask, input_word_len,
           output_word_len, emb, w_ih_enc, w_hh_enc, b_ih_enc, b_hh_enc,
           w_ih_dec, w_hh_dec, b_ih_dec, b_hh_dec):
    in_len = input_word_len[:, 0]
    perm_in = jnp.argsort(-in_len)
    ids_e = jnp.take(input, perm_in, axis=0).astype(jnp.int16)
    len_e = jnp.take(in_len, perm_in).astype(jnp.int32)

    out_len = output_word_len[:, 0]
    perm_out = jnp.argsort(-out_len)
    ids_d = jnp.take(output, perm_out, axis=0).astype(jnp.int16)
    len_d = jnp.take(out_len, perm_out).astype(jnp.int32)

    maxlens = jnp.stack([len_e[::BLK], len_d[::BLK]], axis=1).astype(jnp.int32)

    full = lambda shape: pl.BlockSpec(shape, lambda i, *_: (0,) * len(shape))
    ids_spec = pl.BlockSpec((BLK, L), lambda i, *_: (i, 0))
    len_spec = pl.BlockSpec((BLK, 1), lambda i, *_: (i, 0))

    grid_spec = pltpu.PrefetchScalarGridSpec(
        num_scalar_prefetch=1,
        grid=(NB,),
        in_specs=[
            ids_spec, len_spec, ids_spec, len_spec,
            full((V, E)),
            full((3 * H, E)), full((3 * H, H)), full((1, 3 * H)),
            full((3 * H, E)), full((3 * H, H)), full((1, 3 * H)),
            full((1, 3 * H)), full((1, 3 * H)),
        ],
        out_specs=pl.BlockSpec((BLK, L * H), lambda i, *_: (i, 0)),
        scratch_shapes=[
            pltpu.VMEM((V, 3 * H), jnp.bfloat16),
            pltpu.VMEM((V, 3 * H), jnp.bfloat16),
            pltpu.VMEM((BLK, H), jnp.float32),
        ],
    )

    out_flat = pl.pallas_call(
        _body,
        grid_spec=grid_spec,
        out_shape=jax.ShapeDtypeStruct((B, L * H), jnp.float32),
    )(maxlens, ids_e, len_e[:, None], ids_d, len_d[:, None], emb,
      w_ih_enc, w_hh_enc, b_ih_enc.reshape(1, 3 * H),
      w_ih_dec, w_hh_dec, b_ih_dec.reshape(1, 3 * H),
      b_hh_enc.reshape(1, 3 * H), b_hh_dec.reshape(1, 3 * H))
    return out_flat.reshape(B, L, H)
